# Initial kernel scaffold; baseline (speedup 1.0000x reference)
import functools

import jax
import jax.numpy as jnp
from jax import lax
from jax.experimental import pallas as pl
from jax.experimental.pallas import tpu as pltpu
from jax.experimental.pallas import tpu_sc as plsc

N = 10000
E = 320000
D = 128

NC = 2
NS = 16
NW = NC * NS

C = 80
EDGES_PER_WORKER = E // NW
CHUNKS_PER_WORKER = EDGES_PER_WORKER // C

ROWS_PER_SUB = N // NS
ZROWS = 125

_MM_BLK = 2000


def _matmul_body(x_ref, w_ref, o_ref):
    o_ref[...] = jax.lax.dot_general(
        x_ref[...], w_ref[...], (((1,), (0,)), ((), ())),
        preferred_element_type=jnp.float32,
        precision=jax.lax.Precision.HIGHEST,
    )


def _matmul(x, w):
    return pl.pallas_call(
        _matmul_body,
        grid=(N // _MM_BLK,),
        in_specs=[
            pl.BlockSpec((_MM_BLK, D), lambda i: (i, 0)),
            pl.BlockSpec((D, D), lambda i: (0, 0)),
        ],
        out_specs=pl.BlockSpec((_MM_BLK, D), lambda i: (i, 0)),
        out_shape=jax.ShapeDtypeStruct((N, D), jnp.float32),
    )(x, w)


_sc_mesh = plsc.VectorSubcoreMesh(core_axis_name="c", subcore_axis_name="s")


@functools.partial(
    pl.kernel,
    mesh=_sc_mesh,
    out_type=jax.ShapeDtypeStruct((NC, N, D), jnp.float32),
    scratch_types=[
        pltpu.VMEM((C,), jnp.int32),
        pltpu.VMEM((C,), jnp.int32),
        pltpu.VMEM((C,), jnp.float32),
        pltpu.VMEM((C, D), jnp.float32),
        pltpu.VMEM((ZROWS, D), jnp.float32),
        pltpu.VMEM_SHARED((N, D), jnp.float32),
    ],
)
def _sc_scatter(pre_hbm, rows_hbm, cols_hbm, vals_hbm, out_hbm,
                cols_v, rows_v, vals_v, msg_v, zero_v, acc_sh):
    cid = lax.axis_index("c")
    sid = lax.axis_index("s")
    wid = sid * NC + cid

    @pl.loop(0, ZROWS)
    def _(r):
        @pl.loop(0, D, step=16)
        def _(j):
            zero_v[r, pl.ds(j, 16)] = jnp.zeros((16,), jnp.float32)

    @pl.loop(0, ROWS_PER_SUB, step=ZROWS)
    def _(r0):
        pltpu.sync_copy(zero_v, acc_sh.at[pl.ds(sid * ROWS_PER_SUB + r0, ZROWS)])

    plsc.subcore_barrier()

    base = wid * EDGES_PER_WORKER

    @pl.loop(0, CHUNKS_PER_WORKER)
    def _(i):
        off = base + i * C
        pltpu.sync_copy(cols_hbm.at[pl.ds(off, C)], cols_v)
        pltpu.sync_copy(rows_hbm.at[pl.ds(off, C)], rows_v)
        pltpu.sync_copy(vals_hbm.at[pl.ds(off, C)], vals_v)
        pltpu.sync_copy(pre_hbm.at[cols_v], msg_v)

        @pl.loop(0, C)
        def _(e):
            s = vals_v[e]
            for j in range(0, D, 16):
                msg_v[e, pl.ds(j, 16)] = msg_v[e, pl.ds(j, 16)] * s

        pltpu.sync_copy(msg_v, acc_sh.at[rows_v], add=True)

    plsc.subcore_barrier()

    pltpu.sync_copy(
        acc_sh.at[pl.ds(sid * ROWS_PER_SUB, ROWS_PER_SUB)],
        out_hbm.at[cid, pl.ds(sid * ROWS_PER_SUB, ROWS_PER_SUB)],
    )


def _combine_body(p0_ref, p1_ref, b_ref, o_ref):
    o_ref[...] = jnp.maximum(p0_ref[...] + p1_ref[...] + b_ref[...], 0.0)


def _combine(p0, p1, b2d):
    return pl.pallas_call(
        _combine_body,
        grid=(N // _MM_BLK,),
        in_specs=[
            pl.BlockSpec((_MM_BLK, D), lambda i: (i, 0)),
            pl.BlockSpec((_MM_BLK, D), lambda i: (i, 0)),
            pl.BlockSpec((1, D), lambda i: (0, 0)),
        ],
        out_specs=pl.BlockSpec((_MM_BLK, D), lambda i: (i, 0)),
        out_shape=jax.ShapeDtypeStruct((N, D), jnp.float32),
    )(p0, p1, b2d)


def kernel(x, support_indices, support_values, W0, b):
    pre_sup = _matmul(x, W0)
    rows = support_indices[0]
    cols = support_indices[1]
    partial = _sc_scatter(pre_sup, rows, cols, support_values)
    return _combine(partial[0], partial[1], b.reshape(1, D))


# speedup vs baseline: 4.4321x; 4.4321x over previous
import functools

import jax
import jax.numpy as jnp
from jax import lax
from jax.experimental import pallas as pl
from jax.experimental.pallas import tpu as pltpu
from jax.experimental.pallas import tpu_sc as plsc

N = 10000
E = 320000
D = 128

NC = 2
NS = 16
NW = NC * NS

C = 80
EDGES_PER_WORKER = E // NW
CHUNKS_PER_WORKER = EDGES_PER_WORKER // C

WCHUNK = 200
NWCHUNK = N // WCHUNK

_MM_BLK = 2000


def _matmul_body(x_ref, w_ref, o_ref):
    o_ref[...] = jax.lax.dot_general(
        x_ref[...], w_ref[...], (((1,), (0,)), ((), ())),
        preferred_element_type=jnp.float32,
        precision=jax.lax.Precision.HIGHEST,
    )


def _matmul(x, w):
    return pl.pallas_call(
        _matmul_body,
        grid=(N // _MM_BLK,),
        in_specs=[
            pl.BlockSpec((_MM_BLK, D), lambda i: (i, 0)),
            pl.BlockSpec((D, D), lambda i: (0, 0)),
        ],
        out_specs=pl.BlockSpec((_MM_BLK, D), lambda i: (i, 0)),
        out_shape=jax.ShapeDtypeStruct((N, D), jnp.float32),
    )(x, w)


_sc_mesh = plsc.VectorSubcoreMesh(core_axis_name="c", subcore_axis_name="s")


@functools.partial(
    pl.kernel,
    mesh=_sc_mesh,
    out_type=jax.ShapeDtypeStruct((NC, N, D), jnp.float32),
    scratch_types=[
        pltpu.VMEM((C,), jnp.int32),
        pltpu.VMEM((C,), jnp.int32),
        pltpu.VMEM((C,), jnp.float32),
        pltpu.VMEM((C, D), jnp.float32),
        pltpu.VMEM((WCHUNK, D), jnp.float32),
        pltpu.VMEM_SHARED((N, D), jnp.float32),
    ],
)
def _sc_scatter(pre_hbm, rows_hbm, cols_hbm, vals_hbm, out_hbm,
                cols_v, rows_v, vals_v, msg_v, zero_v, acc_sh):
    cid = lax.axis_index("c")
    sid = lax.axis_index("s")
    wid = sid * NC + cid

    @pl.loop(0, WCHUNK)
    def _(r):
        @pl.loop(0, D, step=16)
        def _(j):
            zero_v[r, pl.ds(j, 16)] = jnp.zeros((16,), jnp.float32)

    @pl.loop(0, NWCHUNK, step=NS)
    def _(t):
        g = t + sid

        @pl.when(g < NWCHUNK)
        def _():
            pltpu.sync_copy(zero_v, acc_sh.at[pl.ds(g * WCHUNK, WCHUNK)])

    plsc.subcore_barrier()

    base = wid * EDGES_PER_WORKER

    @pl.loop(0, CHUNKS_PER_WORKER)
    def _(i):
        off = base + i * C
        pltpu.sync_copy(cols_hbm.at[pl.ds(off, C)], cols_v)
        pltpu.sync_copy(rows_hbm.at[pl.ds(off, C)], rows_v)
        pltpu.sync_copy(vals_hbm.at[pl.ds(off, C)], vals_v)
        pltpu.sync_copy(pre_hbm.at[cols_v], msg_v)

        @pl.loop(0, C, step=16)
        def _(e0):
            v16 = vals_v[pl.ds(e0, 16)]
            for k in range(16):
                s = v16[k]
                for j in range(0, D, 16):
                    msg_v[e0 + k, pl.ds(j, 16)] = msg_v[e0 + k, pl.ds(j, 16)] * s

        pltpu.sync_copy(msg_v, acc_sh.at[rows_v], add=True)

    plsc.subcore_barrier()

    @pl.loop(0, NWCHUNK, step=NS)
    def _(t):
        g = t + sid

        @pl.when(g < NWCHUNK)
        def _():
            pltpu.sync_copy(
                acc_sh.at[pl.ds(g * WCHUNK, WCHUNK)],
                out_hbm.at[cid, pl.ds(g * WCHUNK, WCHUNK)],
            )


def _combine_body(p0_ref, p1_ref, b_ref, o_ref):
    o_ref[...] = jnp.maximum(p0_ref[...] + p1_ref[...] + b_ref[...], 0.0)


def _combine(p0, p1, b2d):
    return pl.pallas_call(
        _combine_body,
        grid=(N // _MM_BLK,),
        in_specs=[
            pl.BlockSpec((_MM_BLK, D), lambda i: (i, 0)),
            pl.BlockSpec((_MM_BLK, D), lambda i: (i, 0)),
            pl.BlockSpec((1, D), lambda i: (0, 0)),
        ],
        out_specs=pl.BlockSpec((_MM_BLK, D), lambda i: (i, 0)),
        out_shape=jax.ShapeDtypeStruct((N, D), jnp.float32),
    )(p0, p1, b2d)


def kernel(x, support_indices, support_values, W0, b):
    pre_sup = _matmul(x, W0)
    rows = support_indices[0]
    cols = support_indices[1]
    partial = _sc_scatter(pre_sup, rows, cols, support_values)
    return _combine(partial[0], partial[1], b.reshape(1, D))


# speedup vs baseline: 9.3950x; 2.1198x over previous
import functools

import jax
import jax.numpy as jnp
from jax import lax
from jax.experimental import pallas as pl
from jax.experimental.pallas import tpu as pltpu
from jax.experimental.pallas import tpu_sc as plsc

N = 10000
E = 320000
D = 128

NC = 2
NS = 16
NW = NC * NS

C = 40
EDGES_PER_WORKER = E // NW
ITEMS = EDGES_PER_WORKER // C
NB = 5
GROUPS = ITEMS // NB

WCHUNK = 80
NWCHUNK = N // WCHUNK

_MM_BLK = 2000


def _matmul_body(x_ref, w_ref, o_ref):
    o_ref[...] = jax.lax.dot_general(
        x_ref[...], w_ref[...], (((1,), (0,)), ((), ())),
        preferred_element_type=jnp.float32,
        precision=jax.lax.Precision.HIGHEST,
    )


def _matmul(x, w):
    return pl.pallas_call(
        _matmul_body,
        grid=(N // _MM_BLK,),
        in_specs=[
            pl.BlockSpec((_MM_BLK, D), lambda i: (i, 0)),
            pl.BlockSpec((D, D), lambda i: (0, 0)),
        ],
        out_specs=pl.BlockSpec((_MM_BLK, D), lambda i: (i, 0)),
        out_shape=jax.ShapeDtypeStruct((N, D), jnp.float32),
    )(x, w)


_sc_mesh = plsc.VectorSubcoreMesh(core_axis_name="c", subcore_axis_name="s")

_SCRATCH = (
    [pltpu.VMEM((C,), jnp.int32) for _ in range(NB)]
    + [pltpu.VMEM((C,), jnp.int32) for _ in range(NB)]
    + [pltpu.VMEM((C,), jnp.float32) for _ in range(NB)]
    + [pltpu.VMEM((C, D), jnp.float32) for _ in range(NB)]
    + [
        pltpu.VMEM((WCHUNK, D), jnp.float32),
        pltpu.VMEM_SHARED((N, D), jnp.float32),
    ]
    + [pltpu.SemaphoreType.DMA for _ in range(3 * NB)]
)


@functools.partial(
    pl.kernel,
    mesh=_sc_mesh,
    out_type=jax.ShapeDtypeStruct((NC, N, D), jnp.float32),
    scratch_types=_SCRATCH,
)
def _sc_scatter(pre_hbm, rows_hbm, cols_hbm, vals_hbm, out_hbm, *scr):
    cols_v = scr[0:NB]
    rows_v = scr[NB:2 * NB]
    vals_v = scr[2 * NB:3 * NB]
    msg = scr[3 * NB:4 * NB]
    zero_v = scr[4 * NB]
    acc_sh = scr[4 * NB + 1]
    isem = scr[4 * NB + 2:4 * NB + 2 + NB]
    gsem = scr[4 * NB + 2 + NB:4 * NB + 2 + 2 * NB]
    asem = scr[4 * NB + 2 + 2 * NB:4 * NB + 2 + 3 * NB]

    cid = lax.axis_index("c")
    sid = lax.axis_index("s")
    wid = sid * NC + cid
    base = wid * EDGES_PER_WORKER

    def idx_start(i, w):
        off = base + i * C
        pltpu.async_copy(cols_hbm.at[pl.ds(off, C)], cols_v[w], isem[w])
        pltpu.async_copy(rows_hbm.at[pl.ds(off, C)], rows_v[w], isem[w])
        pltpu.async_copy(vals_hbm.at[pl.ds(off, C)], vals_v[w], isem[w])

    def idx_wait(w):
        pltpu.make_async_copy(cols_hbm.at[pl.ds(0, C)], cols_v[w], isem[w]).wait()
        pltpu.make_async_copy(rows_hbm.at[pl.ds(0, C)], rows_v[w], isem[w]).wait()
        pltpu.make_async_copy(vals_hbm.at[pl.ds(0, C)], vals_v[w], isem[w]).wait()

    def gather_start(i, w):
        del i
        pltpu.async_copy(pre_hbm.at[cols_v[w]], msg[w], gsem[w])

    def gather_wait(w):
        pltpu.make_async_copy(pre_hbm.at[pl.ds(0, C)], msg[w], gsem[w]).wait()

    def scatter_start(w):
        pltpu.async_copy(msg[w], acc_sh.at[rows_v[w]], asem[w], add=True)

    def scatter_wait(w):
        pltpu.make_async_copy(pre_hbm.at[pl.ds(0, C)], msg[w], asem[w]).wait()

    def scale(w):
        mb = msg[w]
        vb = vals_v[w]
        for e0, k0 in ((0, 0), (16, 0), (24, 8)):
            v16 = vb[pl.ds(e0, 16)]
            for k in range(k0, 16):
                s = v16[k]
                e = e0 + k
                for jj in range(0, D, 16):
                    mb[e, pl.ds(jj, 16)] = mb[e, pl.ds(jj, 16)] * s

    @pl.loop(0, WCHUNK)
    def _(r):
        @pl.loop(0, D, step=16)
        def _(j):
            zero_v[r, pl.ds(j, 16)] = jnp.zeros((16,), jnp.float32)

    @pl.loop(0, NWCHUNK, step=NS)
    def _(t):
        g = t + sid

        @pl.when(g < NWCHUNK)
        def _():
            pltpu.sync_copy(zero_v, acc_sh.at[pl.ds(g * WCHUNK, WCHUNK)])

    for w in range(NB - 1):
        idx_start(w, w)
    for w in range(NB - 2):
        idx_wait(w)
        gather_start(w, w)

    plsc.subcore_barrier()

    @pl.loop(0, GROUPS)
    def _(g):
        for b in range(NB):
            i = g * NB + b
            gather_wait(b)
            scale(b)
            scatter_start(b)

            w4 = (b + 4) % NB
            if b == 0:
                @pl.when(g > 0)
                def _():
                    scatter_wait(w4)

                idx_start(i + 4, w4)
            else:
                @pl.when(g < GROUPS - 1)
                def _():
                    scatter_wait(w4)
                    idx_start(i + 4, w4)

            w3 = (b + 3) % NB
            if b <= 1:
                idx_wait(w3)
                gather_start(i + 3, w3)
            else:
                @pl.when(g < GROUPS - 1)
                def _():
                    idx_wait(w3)
                    gather_start(i + 3, w3)

    for w in range(NB):
        scatter_wait(w)

    plsc.subcore_barrier()

    @pl.loop(0, NWCHUNK, step=NS)
    def _(t):
        g = t + sid

        @pl.when(g < NWCHUNK)
        def _():
            pltpu.sync_copy(
                acc_sh.at[pl.ds(g * WCHUNK, WCHUNK)],
                out_hbm.at[cid, pl.ds(g * WCHUNK, WCHUNK)],
            )


def _combine_body(p0_ref, p1_ref, b_ref, o_ref):
    o_ref[...] = jnp.maximum(p0_ref[...] + p1_ref[...] + b_ref[...], 0.0)


def _combine(p0, p1, b2d):
    return pl.pallas_call(
        _combine_body,
        grid=(N // _MM_BLK,),
        in_specs=[
            pl.BlockSpec((_MM_BLK, D), lambda i: (i, 0)),
            pl.BlockSpec((_MM_BLK, D), lambda i: (i, 0)),
            pl.BlockSpec((1, D), lambda i: (0, 0)),
        ],
        out_specs=pl.BlockSpec((_MM_BLK, D), lambda i: (i, 0)),
        out_shape=jax.ShapeDtypeStruct((N, D), jnp.float32),
    )(p0, p1, b2d)


def kernel(x, support_indices, support_values, W0, b):
    pre_sup = _matmul(x, W0)
    rows = support_indices[0]
    cols = support_indices[1]
    partial = _sc_scatter(pre_sup, rows, cols, support_values)
    return _combine(partial[0], partial[1], b.reshape(1, D))


# speedup vs baseline: 9.7208x; 1.0347x over previous
import functools

import jax
import jax.numpy as jnp
from jax import lax
from jax.experimental import pallas as pl
from jax.experimental.pallas import tpu as pltpu
from jax.experimental.pallas import tpu_sc as plsc

N = 10000
E = 320000
D = 128

NC = 2
NS = 16
NW = NC * NS

C = 40
EDGES_PER_WORKER = E // NW
ITEMS = EDGES_PER_WORKER // C
NB = 5
GROUPS = ITEMS // NB

WCHUNK = 80
NWCHUNK = N // WCHUNK

_MM_BLK = 2000


def _matmul_body(x_ref, w_ref, o_ref):
    o_ref[...] = jax.lax.dot_general(
        x_ref[...], w_ref[...], (((1,), (0,)), ((), ())),
        preferred_element_type=jnp.float32,
        precision=jax.lax.Precision.HIGHEST,
    )


def _matmul(x, w):
    return pl.pallas_call(
        _matmul_body,
        grid=(N // _MM_BLK,),
        in_specs=[
            pl.BlockSpec((_MM_BLK, D), lambda i: (i, 0)),
            pl.BlockSpec((D, D), lambda i: (0, 0)),
        ],
        out_specs=pl.BlockSpec((_MM_BLK, D), lambda i: (i, 0)),
        out_shape=jax.ShapeDtypeStruct((N, D), jnp.float32),
    )(x, w)


_sc_mesh = plsc.VectorSubcoreMesh(core_axis_name="c", subcore_axis_name="s")

_SCRATCH = (
    [pltpu.VMEM((C,), jnp.int32) for _ in range(NB)]
    + [pltpu.VMEM((C,), jnp.int32) for _ in range(NB)]
    + [pltpu.VMEM((C,), jnp.float32) for _ in range(NB)]
    + [pltpu.VMEM((C, D), jnp.float32) for _ in range(NB)]
    + [
        pltpu.VMEM((WCHUNK, D), jnp.float32),
        pltpu.VMEM_SHARED((N, D), jnp.float32),
    ]
    + [pltpu.SemaphoreType.DMA for _ in range(3 * NB)]
)


@functools.partial(
    pl.kernel,
    mesh=_sc_mesh,
    out_type=jax.ShapeDtypeStruct((NC, N, D), jnp.float32),
    scratch_types=_SCRATCH,
)
def _sc_scatter(pre_hbm, rows_hbm, cols_hbm, vals_hbm, out_hbm, *scr):
    cols_v = scr[0:NB]
    rows_v = scr[NB:2 * NB]
    vals_v = scr[2 * NB:3 * NB]
    msg = scr[3 * NB:4 * NB]
    zero_v = scr[4 * NB]
    acc_sh = scr[4 * NB + 1]
    isem = scr[4 * NB + 2:4 * NB + 2 + NB]
    gsem = scr[4 * NB + 2 + NB:4 * NB + 2 + 2 * NB]
    asem = scr[4 * NB + 2 + 2 * NB:4 * NB + 2 + 3 * NB]

    cid = lax.axis_index("c")
    sid = lax.axis_index("s")
    wid = sid * NC + cid
    base = wid * EDGES_PER_WORKER

    def idx_start(i, w):
        off = base + i * C
        pltpu.async_copy(cols_hbm.at[pl.ds(off, C)], cols_v[w], isem[w])
        pltpu.async_copy(rows_hbm.at[pl.ds(off, C)], rows_v[w], isem[w])
        pltpu.async_copy(vals_hbm.at[pl.ds(off, C)], vals_v[w], isem[w])

    def idx_wait(w):
        pltpu.make_async_copy(cols_hbm.at[pl.ds(0, C)], cols_v[w], isem[w]).wait()
        pltpu.make_async_copy(rows_hbm.at[pl.ds(0, C)], rows_v[w], isem[w]).wait()
        pltpu.make_async_copy(vals_hbm.at[pl.ds(0, C)], vals_v[w], isem[w]).wait()

    def gather_start(i, w):
        del i
        pltpu.async_copy(pre_hbm.at[cols_v[w]], msg[w], gsem[w])

    def gather_wait(w):
        pltpu.make_async_copy(pre_hbm.at[pl.ds(0, C)], msg[w], gsem[w]).wait()

    def scatter_start(w):
        pltpu.async_copy(msg[w], acc_sh.at[rows_v[w]], asem[w], add=True)

    def scatter_wait(w):
        pltpu.make_async_copy(pre_hbm.at[pl.ds(0, C)], msg[w], asem[w]).wait()

    def scale(w):
        mb = msg[w]
        vb = vals_v[w]
        for e0, k0 in ((0, 0), (16, 0), (24, 8)):
            v16 = vb[pl.ds(e0, 16)]
            for k in range(k0, 16):
                s = v16[k]
                e = e0 + k
                for jj in range(0, D, 16):
                    mb[e, pl.ds(jj, 16)] = mb[e, pl.ds(jj, 16)] * s

    @pl.loop(0, WCHUNK)
    def _(r):
        @pl.loop(0, D, step=16)
        def _(j):
            zero_v[r, pl.ds(j, 16)] = jnp.zeros((16,), jnp.float32)

    @pl.loop(0, NWCHUNK, step=NS)
    def _(t):
        g = t + sid

        @pl.when(g < NWCHUNK)
        def _():
            pltpu.sync_copy(zero_v, acc_sh.at[pl.ds(g * WCHUNK, WCHUNK)])

    for w in range(NB - 1):
        idx_start(w, w)
    for w in range(NB - 2):
        idx_wait(w)
        gather_start(w, w)

    plsc.subcore_barrier()

    @pl.loop(0, GROUPS)
    def _(g):
        for b in range(NB):
            i = g * NB + b
            gather_wait(b)
            scale(b)
            scatter_start(b)

            w4 = (b + 4) % NB
            if b == 0:
                @pl.when(g > 0)
                def _():
                    scatter_wait(w4)

                idx_start(i + 4, w4)
            else:
                @pl.when(g < GROUPS - 1)
                def _():
                    scatter_wait(w4)
                    idx_start(i + 4, w4)

            w3 = (b + 3) % NB
            if b <= 1:
                idx_wait(w3)
                gather_start(i + 3, w3)
            else:
                @pl.when(g < GROUPS - 1)
                def _():
                    idx_wait(w3)
                    gather_start(i + 3, w3)

    for w in range(NB):
        scatter_wait(w)

    plsc.subcore_barrier()

    @pl.loop(0, NWCHUNK, step=NS)
    def _(t):
        g = t + sid

        @pl.when(g < NWCHUNK)
        def _():
            pltpu.sync_copy(
                acc_sh.at[pl.ds(g * WCHUNK, WCHUNK)],
                out_hbm.at[cid, pl.ds(g * WCHUNK, WCHUNK)],
            )


def _combine_body(p0_ref, p1_ref, b_ref, o_ref):
    o_ref[...] = jnp.maximum(p0_ref[0] + p1_ref[0] + b_ref[...], 0.0)


def _combine(partial, b2d):
    return pl.pallas_call(
        _combine_body,
        grid=(N // _MM_BLK,),
        in_specs=[
            pl.BlockSpec((1, _MM_BLK, D), lambda i: (0, i, 0)),
            pl.BlockSpec((1, _MM_BLK, D), lambda i: (1, i, 0)),
            pl.BlockSpec((1, D), lambda i: (0, 0)),
        ],
        out_specs=pl.BlockSpec((_MM_BLK, D), lambda i: (i, 0)),
        out_shape=jax.ShapeDtypeStruct((N, D), jnp.float32),
    )(partial, partial, b2d)


def kernel(x, support_indices, support_values, W0, b):
    pre_sup = _matmul(x, W0)
    rows = support_indices[0]
    cols = support_indices[1]
    partial = _sc_scatter(pre_sup, rows, cols, support_values)
    return _combine(partial, b.reshape(1, D))
